# Initial kernel scaffold; baseline (speedup 1.0000x reference)
import functools

import jax
import jax.numpy as jnp
from jax import lax
from jax.experimental import pallas as pl
from jax.experimental.pallas import tpu as pltpu
from jax.experimental.pallas import tpu_sc as plsc


@functools.lru_cache(maxsize=None)
def _build_gather(n_total: int, d: int):
    info = plsc.get_sparse_core_info()
    nc, ns = info.num_cores, info.num_subcores
    nw = nc * ns
    assert n_total % nw == 0
    n_per_w = n_total // nw
    chunk = 512
    assert n_per_w % chunk == 0
    n_chunks = n_per_w // chunk
    mesh = plsc.VectorSubcoreMesh(core_axis_name="c", subcore_axis_name="s")

    @functools.partial(
        pl.kernel,
        mesh=mesh,
        out_type=jax.ShapeDtypeStruct((n_total, d), jnp.float32),
        scratch_types=[
            pltpu.VMEM((chunk,), jnp.int32),
            pltpu.VMEM((chunk, d), jnp.float32),
            pltpu.SemaphoreType.DMA,
        ],
    )
    def k(ids_hbm, table_hbm, out_hbm, idx_v, rows_v, sem):
        wid = lax.axis_index("s") * nc + lax.axis_index("c")
        base = wid * n_per_w

        def body(g, carry):
            off = pl.multiple_of(base + g * chunk, chunk)
            pltpu.sync_copy(ids_hbm.at[pl.ds(off, chunk)], idx_v)
            pltpu.async_copy(table_hbm.at[idx_v], rows_v, sem).wait()
            pltpu.sync_copy(rows_v, out_hbm.at[pl.ds(off, chunk)])
            return carry

        lax.fori_loop(0, n_chunks, body, 0)

    return k


def kernel(input_ids, table):
    b, s = input_ids.shape
    d = table.shape[1]
    flat = input_ids.reshape(b * s).astype(jnp.int32)
    out = _build_gather(b * s, d)(flat, table)
    return out.reshape(b, s, d)


# speedup vs baseline: 1.8097x; 1.8097x over previous
import functools

import jax
import jax.numpy as jnp
from jax import lax
from jax.experimental import pallas as pl
from jax.experimental.pallas import tpu as pltpu
from jax.experimental.pallas import tpu_sc as plsc


@functools.lru_cache(maxsize=None)
def _build_gather(n_total: int, d: int):
    info = plsc.get_sparse_core_info()
    nc, ns = info.num_cores, info.num_subcores
    nw = nc * ns
    assert n_total % nw == 0
    n_per_w = n_total // nw
    chunk = 512
    assert n_per_w % chunk == 0
    n_chunks = n_per_w // chunk
    mesh = plsc.VectorSubcoreMesh(core_axis_name="c", subcore_axis_name="s")

    @functools.partial(
        pl.kernel,
        mesh=mesh,
        compiler_params=pltpu.CompilerParams(use_tc_tiling_on_sc=False),
        out_type=jax.ShapeDtypeStruct((n_total, d), jnp.float32),
        scratch_types=[
            pltpu.VMEM((chunk,), jnp.int32),
            pltpu.VMEM((chunk, d), jnp.float32),
            pltpu.SemaphoreType.DMA,
        ],
    )
    def k(ids_hbm, table_hbm, out_hbm, idx_v, rows_v, sem):
        wid = lax.axis_index("s") * nc + lax.axis_index("c")
        base = wid * n_per_w

        def body(g, carry):
            off = pl.multiple_of(base + g * chunk, chunk)
            pltpu.sync_copy(ids_hbm.at[pl.ds(off, chunk)], idx_v)
            pltpu.async_copy(table_hbm.at[idx_v], rows_v, sem).wait()
            pltpu.sync_copy(rows_v, out_hbm.at[pl.ds(off, chunk)])
            return carry

        lax.fori_loop(0, n_chunks, body, 0)

    return k


def kernel(input_ids, table):
    b, s = input_ids.shape
    d = table.shape[1]
    flat = input_ids.reshape(b * s).astype(jnp.int32)
    out = _build_gather(b * s, d)(flat, table)
    return out.reshape(b, s, d)


# speedup vs baseline: 1.8733x; 1.0352x over previous
import functools

import jax
import jax.numpy as jnp
from jax import lax
from jax.experimental import pallas as pl
from jax.experimental.pallas import tpu as pltpu
from jax.experimental.pallas import tpu_sc as plsc

_NBUF = 2
_CHUNK = 512


@functools.lru_cache(maxsize=None)
def _build_gather(n_total: int, d: int):
    info = plsc.get_sparse_core_info()
    nc, ns = info.num_cores, info.num_subcores
    nw = nc * ns
    assert n_total % nw == 0
    n_per_w = n_total // nw
    chunk = _CHUNK
    nbuf = _NBUF
    assert n_per_w % (chunk * nbuf) == 0
    n_chunks = n_per_w // chunk
    outer = n_chunks // nbuf
    mesh = plsc.VectorSubcoreMesh(core_axis_name="c", subcore_axis_name="s")

    @functools.partial(
        pl.kernel,
        mesh=mesh,
        compiler_params=pltpu.CompilerParams(use_tc_tiling_on_sc=False),
        out_type=jax.ShapeDtypeStruct((n_total, d), jnp.float32),
        scratch_types=[
            pltpu.VMEM((n_per_w,), jnp.int32),
            pltpu.VMEM((nbuf, chunk, d), jnp.float32),
            pltpu.SemaphoreType.DMA((nbuf,)),
            pltpu.SemaphoreType.DMA((nbuf,)),
        ],
    )
    def k(ids_hbm, table_hbm, out_hbm, idx_v, rows_v, gsem, wsem):
        wid = lax.axis_index("s") * nc + lax.axis_index("c")
        base = wid * n_per_w

        pltpu.sync_copy(ids_hbm.at[pl.ds(pl.multiple_of(base, chunk), n_per_w)],
                        idx_v)

        def start_gather(g, b):
            loc = pl.multiple_of(g * chunk, chunk)
            pltpu.async_copy(
                table_hbm.at[idx_v.at[pl.ds(loc, chunk)]],
                rows_v.at[b], gsem.at[b])

        def wait_gather(b):
            pltpu.make_async_copy(table_hbm.at[idx_v.at[pl.ds(0, chunk)]],
                                  rows_v.at[b], gsem.at[b]).wait()

        def start_write(g, b):
            off = pl.multiple_of(base + g * chunk, chunk)
            pltpu.async_copy(rows_v.at[b], out_hbm.at[pl.ds(off, chunk)],
                             wsem.at[b])

        def wait_write(b):
            pltpu.make_async_copy(rows_v.at[b],
                                  out_hbm.at[pl.ds(0, chunk)],
                                  wsem.at[b]).wait()

        for b in range(nbuf):
            start_gather(b, b)

        def body(i, carry):
            for b in range(nbuf):
                g = i * nbuf + b
                wait_gather(b)
                start_write(g, b)
                wait_write(b)
                start_gather(g + nbuf, b)
            return carry

        lax.fori_loop(0, outer - 1, body, 0)

        for b in range(nbuf):
            g = (outer - 1) * nbuf + b
            wait_gather(b)
            start_write(g, b)
        for b in range(nbuf):
            wait_write(b)

    return k


def kernel(input_ids, table):
    b, s = input_ids.shape
    d = table.shape[1]
    flat = input_ids.reshape(b * s).astype(jnp.int32)
    out = _build_gather(b * s, d)(flat, table)
    return out.reshape(b, s, d)


# speedup vs baseline: 1.8761x; 1.0015x over previous
import functools

import jax
import jax.numpy as jnp
from jax import lax
from jax.experimental import pallas as pl
from jax.experimental.pallas import tpu as pltpu
from jax.experimental.pallas import tpu_sc as plsc

_NBUF = 2
_CHUNK = 512
_NSTREAM = 4


@functools.lru_cache(maxsize=None)
def _build_gather(n_total: int, d: int):
    info = plsc.get_sparse_core_info()
    nc, ns = info.num_cores, info.num_subcores
    nw = nc * ns
    assert n_total % nw == 0
    n_per_w = n_total // nw
    chunk = _CHUNK
    nbuf = _NBUF
    assert n_per_w % (chunk * nbuf) == 0
    n_chunks = n_per_w // chunk
    outer = n_chunks // nbuf
    mesh = plsc.VectorSubcoreMesh(core_axis_name="c", subcore_axis_name="s")

    @functools.partial(
        pl.kernel,
        mesh=mesh,
        compiler_params=pltpu.CompilerParams(use_tc_tiling_on_sc=False),
        out_type=jax.ShapeDtypeStruct((n_total, d), jnp.float32),
        scratch_types=[
            pltpu.VMEM((n_per_w,), jnp.int32),
            pltpu.VMEM((nbuf, chunk, d), jnp.float32),
            pltpu.SemaphoreType.DMA((nbuf,)),
            pltpu.SemaphoreType.DMA((nbuf,)),
        ],
    )
    def k(ids_hbm, table_hbm, out_hbm, idx_v, rows_v, gsem, wsem):
        wid = lax.axis_index("s") * nc + lax.axis_index("c")
        base = wid * n_per_w

        pltpu.sync_copy(ids_hbm.at[pl.ds(pl.multiple_of(base, chunk), n_per_w)],
                        idx_v)

        sub = chunk // _NSTREAM

        def start_gather(g, b):
            for s in range(_NSTREAM):
                loc = pl.multiple_of(g * chunk + s * sub, sub)
                pltpu.async_copy(
                    table_hbm.at[idx_v.at[pl.ds(loc, sub)]],
                    rows_v.at[b].at[pl.ds(s * sub, sub)], gsem.at[b])

        def wait_gather(b):
            for s in range(_NSTREAM):
                pltpu.make_async_copy(
                    table_hbm.at[idx_v.at[pl.ds(0, sub)]],
                    rows_v.at[b].at[pl.ds(0, sub)], gsem.at[b]).wait()

        def start_write(g, b):
            off = pl.multiple_of(base + g * chunk, chunk)
            pltpu.async_copy(rows_v.at[b], out_hbm.at[pl.ds(off, chunk)],
                             wsem.at[b])

        def wait_write(b):
            pltpu.make_async_copy(rows_v.at[b],
                                  out_hbm.at[pl.ds(0, chunk)],
                                  wsem.at[b]).wait()

        for b in range(nbuf):
            start_gather(b, b)

        def body(i, carry):
            for b in range(nbuf):
                g = i * nbuf + b
                wait_gather(b)
                start_write(g, b)
                wait_write(b)
                start_gather(g + nbuf, b)
            return carry

        lax.fori_loop(0, outer - 1, body, 0)

        for b in range(nbuf):
            g = (outer - 1) * nbuf + b
            wait_gather(b)
            start_write(g, b)
        for b in range(nbuf):
            wait_write(b)

    return k


def kernel(input_ids, table):
    b, s = input_ids.shape
    d = table.shape[1]
    flat = input_ids.reshape(b * s).astype(jnp.int32)
    out = _build_gather(b * s, d)(flat, table)
    return out.reshape(b, s, d)


# speedup vs baseline: 1.8771x; 1.0005x over previous
import functools

import jax
import jax.numpy as jnp
from jax import lax
from jax.experimental import pallas as pl
from jax.experimental.pallas import tpu as pltpu
from jax.experimental.pallas import tpu_sc as plsc

_NBUF = 2
_CHUNK = 512
_ROW = 128


@functools.lru_cache(maxsize=None)
def _build_gather(n_total: int, d: int):
    info = plsc.get_sparse_core_info()
    nc, ns = info.num_cores, info.num_subcores
    nw = nc * ns
    assert n_total % nw == 0
    n_per_w = n_total // nw
    chunk = _CHUNK
    nbuf = _NBUF
    nstream = chunk // _ROW
    assert n_per_w % (chunk * nbuf) == 0
    n_chunks = n_per_w // chunk
    outer = n_chunks // nbuf
    idx_rows_w = n_per_w // _ROW
    mesh = plsc.VectorSubcoreMesh(core_axis_name="c", subcore_axis_name="s")

    @functools.partial(
        pl.kernel,
        mesh=mesh,
        compiler_params=pltpu.CompilerParams(use_tc_tiling_on_sc=False),
        out_type=jax.ShapeDtypeStruct((n_total, d), jnp.float32),
        scratch_types=[
            pltpu.VMEM((idx_rows_w, _ROW), jnp.int32),
            pltpu.VMEM((nbuf, chunk, d), jnp.float32),
            pltpu.SemaphoreType.DMA((nbuf,)),
            pltpu.SemaphoreType.DMA((nbuf,)),
        ],
    )
    def k(ids_hbm, table_hbm, out_hbm, idx_v, rows_v, gsem, wsem):
        wid = lax.axis_index("s") * nc + lax.axis_index("c")
        base = wid * n_per_w

        pltpu.sync_copy(ids_hbm.at[pl.ds(wid * idx_rows_w, idx_rows_w)], idx_v)

        def start_gather(g, b):
            for s in range(nstream):
                r = g * nstream + s
                pltpu.async_copy(
                    table_hbm.at[idx_v.at[r]],
                    rows_v.at[b].at[pl.ds(s * _ROW, _ROW)], gsem.at[b])

        def wait_gather(b):
            for s in range(nstream):
                pltpu.make_async_copy(
                    table_hbm.at[idx_v.at[0]],
                    rows_v.at[b].at[pl.ds(0, _ROW)], gsem.at[b]).wait()

        def start_write(g, b):
            off = pl.multiple_of(base + g * chunk, chunk)
            pltpu.async_copy(rows_v.at[b], out_hbm.at[pl.ds(off, chunk)],
                             wsem.at[b])

        def wait_write(b):
            pltpu.make_async_copy(rows_v.at[b],
                                  out_hbm.at[pl.ds(0, chunk)],
                                  wsem.at[b]).wait()

        for b in range(nbuf):
            start_gather(b, b)

        def body(i, carry):
            for b in range(nbuf):
                g = i * nbuf + b
                wait_gather(b)
                start_write(g, b)
                wait_write(b)
                start_gather(g + nbuf, b)
            return carry

        lax.fori_loop(0, outer - 1, body, 0)

        for b in range(nbuf):
            g = (outer - 1) * nbuf + b
            wait_gather(b)
            start_write(g, b)
        for b in range(nbuf):
            wait_write(b)

    return k


def kernel(input_ids, table):
    b, s = input_ids.shape
    d = table.shape[1]
    n = b * s
    ids2d = input_ids.reshape(n // _ROW, _ROW).astype(jnp.int32)
    out = _build_gather(n, d)(ids2d, table)
    return out.reshape(b, s, d)
